# baseline (speedup 1.0000x reference)
import functools

import jax
import jax.numpy as jnp
from jax import lax
from jax.experimental import pallas as pl
from jax.experimental.pallas import tpu as pltpu
from jax.experimental.pallas import tpu_sc as plsc

N = 10000
E = 320000
D = 128
H = 128
B = 64
NCL = 1000
CED = 64
LAT = 978

NW = 32
NP = 10240
EW = 10240
EP = NW * EW
CHUNK = 128
NSET = 2
NCHUNK = EW // CHUNK
NIROW = EW // 128
DCHUNK = 512
DNCHUNK = EW // DCHUNK
DNI = DNCHUNK // 2
TROWS = NP // 16
RB = 1024
NTB = NP // RB
LATP = 1024

_mesh = plsc.VectorSubcoreMesh(core_axis_name="c", subcore_axis_name="s")


@functools.partial(
    pl.kernel,
    out_type=[
        jax.ShapeDtypeStruct((NW, NP), jnp.float32),
        jax.ShapeDtypeStruct((B, H), jnp.float32),
    ],
    mesh=_mesh,
    scratch_types=[
        pltpu.VMEM((NP,), jnp.float32),
        pltpu.VMEM((DCHUNK,), jnp.int32),
        pltpu.VMEM((DCHUNK,), jnp.int32),
        pltpu.VMEM((B,), jnp.int32),
        pltpu.VMEM((B, H), jnp.float32),
        [pltpu.SemaphoreType.DMA for _ in range(3)],
    ],
    compiler_params=pltpu.CompilerParams(needs_layout_passes=False),
)
def _sc_degree(cols_hbm, emb_hbm, cl_hbm, deg_hbm, ce_hbm,
               histo, idxA, idxB, cl_v, ce_v, sems):
    c = lax.axis_index("c")
    s = lax.axis_index("s")
    w = c * 16 + s
    semA, semB, semE = sems

    def cstart(j, buf, sem):
        pltpu.async_copy(cols_hbm.at[pl.ds(w * EW + j * DCHUNK, DCHUNK)],
                         buf, sem)

    def cwait(buf, sem):
        pltpu.make_async_copy(cols_hbm.at[pl.ds(0, DCHUNK)], buf, sem).wait()

    cstart(0, idxA, semA)
    cstart(1, idxB, semB)

    def _zero(i, carry):
        histo[pl.ds(i * 16, 16)] = jnp.zeros((16,), jnp.float32)
        return carry
    lax.fori_loop(0, NP // 16, _zero, 0)

    ones16 = jnp.ones((16,), jnp.float32)

    def _step(i, carry):
        cwait(idxA, semA)
        for t in range(DCHUNK // 16):
            plsc.addupdate_scatter(histo, [idxA[pl.ds(t * 16, 16)]], ones16)

        @pl.when(i < DNI - 1)
        def _():
            cstart(2 * i + 2, idxA, semA)
        cwait(idxB, semB)
        for t in range(DCHUNK // 16):
            plsc.addupdate_scatter(histo, [idxB[pl.ds(t * 16, 16)]], ones16)

        @pl.when(i < DNI - 1)
        def _():
            cstart(2 * i + 3, idxB, semB)
        return carry
    lax.fori_loop(0, DNI, _step, 0)

    pltpu.sync_copy(histo, deg_hbm.at[w])

    @pl.when(w == 0)
    def _():
        pltpu.sync_copy(cl_hbm, cl_v)
        pltpu.async_copy(emb_hbm.at[cl_v], ce_v, semE).wait()
        pltpu.sync_copy(ce_v, ce_hbm)


NI = NCHUNK // NSET


@functools.partial(
    pl.kernel,
    out_type=jax.ShapeDtypeStruct((2, NP, H), jnp.float32),
    mesh=_mesh,
    scratch_types=[
        pltpu.VMEM_SHARED((NP, H), jnp.float32),
        pltpu.VMEM((NIROW, 128), jnp.int32),
        [pltpu.VMEM((CHUNK,), jnp.int32) for _ in range(NSET)],
        [pltpu.VMEM((CHUNK, H), jnp.float32) for _ in range(NSET)],
        [pltpu.SemaphoreType.DMA for _ in range(3 * NSET)],
    ],
)
def _sc_edge_pass(hp_hbm, rows_hbm, cols_hbm, acc_hbm,
                  acc_sp, idx_r, cbufs, gbufs, sems):
    c = lax.axis_index("c")
    s = lax.axis_index("s")
    w = c * 16 + s
    semg, sems_, semc = sems[:NSET], sems[NSET:2 * NSET], sems[2 * NSET:]

    pltpu.sync_copy(rows_hbm.at[pl.ds(w * NIROW, NIROW)], idx_r)

    z16 = jnp.zeros((16,), jnp.float32)

    def _fill(r, carry):
        for t in range(H // 16):
            gbufs[0][r, pl.ds(t * 16, 16)] = z16
        return carry
    lax.fori_loop(0, CHUNK, _fill, 0)

    def _zero(m, carry):
        pltpu.sync_copy(gbufs[0],
                        acc_sp.at[pl.ds(s * TROWS + m * CHUNK, CHUNK)])
        return carry
    lax.fori_loop(0, TROWS // CHUNK, _zero, 0)

    plsc.subcore_barrier()

    def gidx(i, b):
        row = (NSET * CHUNK // 128) * i + (b * CHUNK) // 128
        return idx_r.at[row, pl.ds((b * CHUNK) % 128, CHUNK)]

    def gstart(i, b):
        pltpu.async_copy(hp_hbm.at[gidx(i, b)], gbufs[b], semg[b])

    def gwait(b):
        pltpu.make_async_copy(hp_hbm.at[gidx(0, 0)], gbufs[b],
                              semg[b]).wait()

    def sstart(b):
        pltpu.async_copy(gbufs[b], acc_sp.at[cbufs[b]], sems_[b], add=True)

    def swait(b):
        pltpu.make_async_copy(gbufs[b], acc_sp.at[cbufs[b]], sems_[b]).wait()

    def cstart(i, b):
        k = NSET * i + b
        pltpu.async_copy(cols_hbm.at[pl.ds(w * EW + k * CHUNK, CHUNK)],
                         cbufs[b], semc[b])

    def cwait(b):
        pltpu.make_async_copy(cols_hbm.at[pl.ds(0, CHUNK)], cbufs[b],
                              semc[b]).wait()

    for b in range(NSET):
        cstart(0, b)
        gstart(0, b)

    def _body(i, carry):
        for b in range(NSET):
            gwait(b)
            cwait(b)
            sstart(b)
        for b in range(NSET):
            swait(b)

            @pl.when(i < NI - 1)
            def _(b=b):
                cstart(i + 1, b)
                gstart(i + 1, b)
        return carry
    lax.fori_loop(0, NI, _body, 0)

    plsc.subcore_barrier()
    pltpu.sync_copy(acc_sp.at[pl.ds(s * TROWS, TROWS)],
                    acc_hbm.at[c, pl.ds(s * TROWS, TROWS)])


def _tc_mm1(x_ref, w1_ref, z_ref):
    z_ref[...] = jnp.dot(x_ref[...], w1_ref[...],
                         preferred_element_type=jnp.float32)


def _tc_scale_in(z_ref, degt_ref, hp_ref, dinvb_ref):
    deg = jnp.sum(degt_ref[...], axis=1, keepdims=True) + 1.0
    dinvb = jnp.broadcast_to(lax.rsqrt(deg), (RB, H))
    hp_ref[...] = dinvb * z_ref[...]
    dinvb_ref[...] = dinvb


def _tc_combine_mm(acc_ref, hp_ref, dinvb_ref, b_ref, w2_ref, hp2_ref):
    dinvb = dinvb_ref[...]
    u = jnp.maximum(
        dinvb * (acc_ref[0] + acc_ref[1] + hp_ref[...]) + b_ref[...], 0.0)
    hp2_ref[...] = dinvb * jnp.dot(u, w2_ref[...],
                                   preferred_element_type=jnp.float32)


def _ln(x, w, b, eps=1e-5):
    mu = jnp.mean(x, axis=-1, keepdims=True)
    var = jnp.mean((x - mu) ** 2, axis=-1, keepdims=True)
    return (x - mu) / jnp.sqrt(var + eps) * w + b


def _tc_pool_head(acc_ref, hp_ref, dinvb_ref, b_ref, batchb_ref, cep_ref,
                  combA_ref, combB_ref, comb_b_ref, lnc_w_ref, lnc_b_ref,
                  fc1_W_ref, fc1_b_ref, ln1_w_ref, ln1_b_ref, fc2_W_ref,
                  fc2_b_ref, out_ref, g_ref):
    h2 = jnp.maximum(
        dinvb_ref[...] * (acc_ref[0] + acc_ref[1] + hp_ref[...]) + b_ref[...],
        0.0)
    onehot = (batchb_ref[...] ==
              lax.broadcasted_iota(jnp.int32, (RB, B), 1)).astype(jnp.float32)
    part = lax.dot_general(onehot, h2, (((0,), (0,)), ((), ())),
                           preferred_element_type=jnp.float32)

    @pl.when(pl.program_id(0) == 0)
    def _():
        g_ref[...] = jnp.zeros_like(g_ref)
    g_ref[...] += part

    @pl.when(pl.program_id(0) == NTB - 1)
    def _():
        v = (jnp.dot(g_ref[...], combA_ref[...],
                     preferred_element_type=jnp.float32)
             + jnp.dot(cep_ref[...], combB_ref[...],
                       preferred_element_type=jnp.float32)
             + comb_b_ref[...])
        c1 = jnp.maximum(_ln(v, lnc_w_ref[...], lnc_b_ref[...]), 0.0)
        o = jnp.maximum(
            jnp.dot(c1, fc1_W_ref[...], preferred_element_type=jnp.float32)
            + fc1_b_ref[...], 0.0)
        o = _ln(o, ln1_w_ref[...], ln1_b_ref[...])
        out_ref[...] = (jnp.dot(o, fc2_W_ref[...],
                                preferred_element_type=jnp.float32)
                        + fc2_b_ref[...])


def _row_spec(nd=H):
    return pl.BlockSpec((RB, nd), lambda i: (i, 0))


def _rep_spec(shape):
    n = len(shape)
    return pl.BlockSpec(shape, lambda i, _n=n: (0,) * _n)


def kernel(x, edge_index, batch, cell_lines, gcn1_W, gcn1_b, gcn2_W, gcn2_b,
           emb, comb_W, comb_b, lnc_w, lnc_b, fc1_W, fc1_b, ln1_w, ln1_b,
           fc2_W, fc2_b):
    f32 = jnp.float32
    xp = jnp.pad(x, ((0, NP - N), (0, 0)))
    pad_idx = N + jnp.arange(EP - E, dtype=jnp.int32) % (NP - N)
    rows = jnp.concatenate([edge_index[0], pad_idx]).reshape(NW * NIROW, 128)
    cols1 = jnp.concatenate([edge_index[1], pad_idx])
    batchp = jnp.concatenate([batch, jnp.full((NP - N,), B, jnp.int32)])
    batchb = jnp.broadcast_to(batchp[:, None], (NP, B))

    embp = jnp.pad(emb, ((0, 0), (0, H - CED)))
    degp, cep = _sc_degree(cols1, embp, cell_lines)
    degt = degp.T

    z1 = pl.pallas_call(
        _tc_mm1,
        grid=(NTB,),
        in_specs=[_row_spec(), _rep_spec((D, H))],
        out_specs=_row_spec(),
        out_shape=jax.ShapeDtypeStruct((NP, H), f32),
    )(xp, gcn1_W)

    hp1, dinvb = pl.pallas_call(
        _tc_scale_in,
        grid=(NTB,),
        in_specs=[_row_spec(), pl.BlockSpec((RB, NW), lambda i: (i, 0))],
        out_specs=[_row_spec(), _row_spec()],
        out_shape=[jax.ShapeDtypeStruct((NP, H), f32)] * 2,
    )(z1, degt)

    acc1 = _sc_edge_pass(hp1, rows, cols1)

    hp2 = pl.pallas_call(
        _tc_combine_mm,
        grid=(NTB,),
        in_specs=[pl.BlockSpec((2, RB, H), lambda i: (0, i, 0)),
                  _row_spec(), _row_spec(), _rep_spec((1, H)),
                  _rep_spec((H, H))],
        out_specs=_row_spec(),
        out_shape=jax.ShapeDtypeStruct((NP, H), f32),
    )(acc1, hp1, dinvb, gcn1_b[None, :], gcn2_W)

    acc2 = _sc_edge_pass(hp2, rows, cols1)

    combA = comb_W[:H]
    combB = jnp.pad(comb_W[H:], ((0, H - CED), (0, 0)))
    fc2_Wp = jnp.pad(fc2_W, ((0, 0), (0, LATP - LAT)))
    fc2_bp = jnp.pad(fc2_b, ((0, LATP - LAT),))

    out = pl.pallas_call(
        _tc_pool_head,
        grid=(NTB,),
        in_specs=[pl.BlockSpec((2, RB, H), lambda i: (0, i, 0)),
                  _row_spec(), _row_spec(), _rep_spec((1, H)),
                  pl.BlockSpec((RB, B), lambda i: (i, 0))] +
                 [_rep_spec(s) for s in
                  [(B, H), (H, H), (H, H), (1, H), (1, H), (1, H),
                   (H, H), (1, H), (1, H), (1, H), (H, LATP), (1, LATP)]],
        out_specs=pl.BlockSpec((B, LATP), lambda i: (0, 0)),
        out_shape=jax.ShapeDtypeStruct((B, LATP), f32),
        scratch_shapes=[pltpu.VMEM((B, H), f32)],
    )(acc2, hp2, dinvb, gcn2_b[None, :], batchb, cep, combA, combB,
      comb_b[None, :], lnc_w[None, :], lnc_b[None, :], fc1_W, fc1_b[None, :],
      ln1_w[None, :], ln1_b[None, :], fc2_Wp, fc2_bp[None, :])

    return out[:, :LAT]


# speedup vs baseline: 1.0033x; 1.0033x over previous
import functools

import jax
import jax.numpy as jnp
from jax import lax
from jax.experimental import pallas as pl
from jax.experimental.pallas import tpu as pltpu
from jax.experimental.pallas import tpu_sc as plsc

N = 10000
E = 320000
D = 128
H = 128
B = 64
NCL = 1000
CED = 64
LAT = 978

NW = 32
NP = 10240
EW = 10240
EP = NW * EW
CHUNK = 128
NSET = 2
NCHUNK = EW // CHUNK
NIROW = EW // 128
DCHUNK = 512
DNCHUNK = EW // DCHUNK
DNI = DNCHUNK // 2
TROWS = NP // 16
RB = 1024
NTB = NP // RB
LATP = 1024

_mesh = plsc.VectorSubcoreMesh(core_axis_name="c", subcore_axis_name="s")


@functools.partial(
    pl.kernel,
    out_type=[
        jax.ShapeDtypeStruct((NW, NP), jnp.float32),
        jax.ShapeDtypeStruct((B, H), jnp.float32),
    ],
    mesh=_mesh,
    scratch_types=[
        pltpu.VMEM((NP,), jnp.float32),
        pltpu.VMEM((DCHUNK,), jnp.int32),
        pltpu.VMEM((DCHUNK,), jnp.int32),
        pltpu.VMEM((B,), jnp.int32),
        pltpu.VMEM((B, H), jnp.float32),
        [pltpu.SemaphoreType.DMA for _ in range(3)],
    ],
    compiler_params=pltpu.CompilerParams(needs_layout_passes=False),
)
def _sc_degree(cols_hbm, emb_hbm, cl_hbm, deg_hbm, ce_hbm,
               histo, idxA, idxB, cl_v, ce_v, sems):
    c = lax.axis_index("c")
    s = lax.axis_index("s")
    w = c * 16 + s
    semA, semB, semE = sems

    def cstart(j, buf, sem):
        pltpu.async_copy(cols_hbm.at[pl.ds(w * EW + j * DCHUNK, DCHUNK)],
                         buf, sem)

    def cwait(buf, sem):
        pltpu.make_async_copy(cols_hbm.at[pl.ds(0, DCHUNK)], buf, sem).wait()

    cstart(0, idxA, semA)
    cstart(1, idxB, semB)

    def _zero(i, carry):
        histo[pl.ds(i * 16, 16)] = jnp.zeros((16,), jnp.float32)
        return carry
    lax.fori_loop(0, NP // 16, _zero, 0)

    ones16 = jnp.ones((16,), jnp.float32)

    def _step(i, carry):
        cwait(idxA, semA)
        for t in range(DCHUNK // 16):
            plsc.addupdate_scatter(histo, [idxA[pl.ds(t * 16, 16)]], ones16)

        @pl.when(i < DNI - 1)
        def _():
            cstart(2 * i + 2, idxA, semA)
        cwait(idxB, semB)
        for t in range(DCHUNK // 16):
            plsc.addupdate_scatter(histo, [idxB[pl.ds(t * 16, 16)]], ones16)

        @pl.when(i < DNI - 1)
        def _():
            cstart(2 * i + 3, idxB, semB)
        return carry
    lax.fori_loop(0, DNI, _step, 0)

    pltpu.sync_copy(histo, deg_hbm.at[w])

    @pl.when(w == 0)
    def _():
        pltpu.sync_copy(cl_hbm, cl_v)
        pltpu.async_copy(emb_hbm.at[cl_v], ce_v, semE).wait()
        pltpu.sync_copy(ce_v, ce_hbm)


NI = NCHUNK // NSET


@functools.partial(
    pl.kernel,
    out_type=jax.ShapeDtypeStruct((2, NP, H), jnp.float32),
    mesh=_mesh,
    scratch_types=[
        pltpu.VMEM_SHARED((NP, H), jnp.float32),
        pltpu.VMEM((NIROW, 128), jnp.int32),
        [pltpu.VMEM((CHUNK,), jnp.int32) for _ in range(NSET)],
        [pltpu.VMEM((CHUNK, H), jnp.float32) for _ in range(NSET)],
        [pltpu.SemaphoreType.DMA for _ in range(3 * NSET)],
    ],
)
def _sc_edge_pass(hp_hbm, rows_hbm, cols_hbm, acc_hbm,
                  acc_sp, idx_r, cbufs, gbufs, sems):
    c = lax.axis_index("c")
    s = lax.axis_index("s")
    w = c * 16 + s
    semg, sems_, semc = sems[:NSET], sems[NSET:2 * NSET], sems[2 * NSET:]

    pltpu.sync_copy(rows_hbm.at[pl.ds(w * NIROW, NIROW)], idx_r)

    z16 = jnp.zeros((16,), jnp.float32)

    def _fill(r, carry):
        for t in range(H // 16):
            gbufs[0][r, pl.ds(t * 16, 16)] = z16
        return carry
    lax.fori_loop(0, CHUNK, _fill, 0)

    def _zero(m, carry):
        pltpu.sync_copy(gbufs[0],
                        acc_sp.at[pl.ds(s * TROWS + m * CHUNK, CHUNK)])
        return carry
    lax.fori_loop(0, TROWS // CHUNK, _zero, 0)

    plsc.subcore_barrier()

    def gidx(i, b):
        row = (NSET * CHUNK // 128) * i + (b * CHUNK) // 128
        return idx_r.at[row, pl.ds((b * CHUNK) % 128, CHUNK)]

    def gstart(i, b):
        pltpu.async_copy(hp_hbm.at[gidx(i, b)], gbufs[b], semg[b])

    def gwait(b):
        pltpu.make_async_copy(hp_hbm.at[gidx(0, 0)], gbufs[b],
                              semg[b]).wait()

    def sstart(b):
        pltpu.async_copy(gbufs[b], acc_sp.at[cbufs[b]], sems_[b], add=True)

    def swait(b):
        pltpu.make_async_copy(gbufs[b], acc_sp.at[cbufs[b]], sems_[b]).wait()

    def cstart(i, b):
        k = NSET * i + b
        pltpu.async_copy(cols_hbm.at[pl.ds(w * EW + k * CHUNK, CHUNK)],
                         cbufs[b], semc[b])

    def cwait(b):
        pltpu.make_async_copy(cols_hbm.at[pl.ds(0, CHUNK)], cbufs[b],
                              semc[b]).wait()

    for b in range(NSET):
        cstart(0, b)
        gstart(0, b)

    def _body(i, carry):
        for b in range(NSET):
            gwait(b)
            cwait(b)
            sstart(b)
        for b in range(NSET):
            swait(b)

            @pl.when(i < NI - 1)
            def _(b=b):
                cstart(i + 1, b)
                gstart(i + 1, b)
        return carry
    lax.fori_loop(0, NI, _body, 0)

    plsc.subcore_barrier()
    pltpu.sync_copy(acc_sp.at[pl.ds(s * TROWS, TROWS)],
                    acc_hbm.at[c, pl.ds(s * TROWS, TROWS)])


def _tc_scale_in(x_ref, degt_ref, w1_ref, hp_ref, dinvb_ref):
    deg = jnp.sum(degt_ref[...], axis=1, keepdims=True) + 1.0
    dinvb = jnp.broadcast_to(lax.rsqrt(deg), (RB, H))
    z = jnp.dot(x_ref[...], w1_ref[...], preferred_element_type=jnp.float32)
    hp_ref[...] = dinvb * z
    dinvb_ref[...] = dinvb


def _tc_combine_mm(acc_ref, hp_ref, dinvb_ref, b_ref, w2_ref, hp2_ref):
    dinvb = dinvb_ref[...]
    u = jnp.maximum(
        dinvb * (acc_ref[0] + acc_ref[1] + hp_ref[...]) + b_ref[...], 0.0)
    hp2_ref[...] = dinvb * jnp.dot(u, w2_ref[...],
                                   preferred_element_type=jnp.float32)


def _ln(x, w, b, eps=1e-5):
    mu = jnp.mean(x, axis=-1, keepdims=True)
    var = jnp.mean((x - mu) ** 2, axis=-1, keepdims=True)
    return (x - mu) / jnp.sqrt(var + eps) * w + b


def _tc_pool_head(acc_ref, hp_ref, dinvb_ref, b_ref, batchb_ref, cep_ref,
                  combA_ref, combB_ref, comb_b_ref, lnc_w_ref, lnc_b_ref,
                  fc1_W_ref, fc1_b_ref, ln1_w_ref, ln1_b_ref, fc2_W_ref,
                  fc2_b_ref, out_ref, g_ref):
    h2 = jnp.maximum(
        dinvb_ref[...] * (acc_ref[0] + acc_ref[1] + hp_ref[...]) + b_ref[...],
        0.0)
    onehot = (batchb_ref[...] ==
              lax.broadcasted_iota(jnp.int32, (RB, B), 1)).astype(jnp.float32)
    part = lax.dot_general(onehot, h2, (((0,), (0,)), ((), ())),
                           preferred_element_type=jnp.float32)

    @pl.when(pl.program_id(0) == 0)
    def _():
        g_ref[...] = jnp.zeros_like(g_ref)
    g_ref[...] += part

    @pl.when(pl.program_id(0) == NTB - 1)
    def _():
        v = (jnp.dot(g_ref[...], combA_ref[...],
                     preferred_element_type=jnp.float32)
             + jnp.dot(cep_ref[...], combB_ref[...],
                       preferred_element_type=jnp.float32)
             + comb_b_ref[...])
        c1 = jnp.maximum(_ln(v, lnc_w_ref[...], lnc_b_ref[...]), 0.0)
        o = jnp.maximum(
            jnp.dot(c1, fc1_W_ref[...], preferred_element_type=jnp.float32)
            + fc1_b_ref[...], 0.0)
        o = _ln(o, ln1_w_ref[...], ln1_b_ref[...])
        out_ref[...] = (jnp.dot(o, fc2_W_ref[...],
                                preferred_element_type=jnp.float32)
                        + fc2_b_ref[...])


def _row_spec(nd=H):
    return pl.BlockSpec((RB, nd), lambda i: (i, 0))


def _rep_spec(shape):
    n = len(shape)
    return pl.BlockSpec(shape, lambda i, _n=n: (0,) * _n)


def kernel(x, edge_index, batch, cell_lines, gcn1_W, gcn1_b, gcn2_W, gcn2_b,
           emb, comb_W, comb_b, lnc_w, lnc_b, fc1_W, fc1_b, ln1_w, ln1_b,
           fc2_W, fc2_b):
    f32 = jnp.float32
    xp = jnp.pad(x, ((0, NP - N), (0, 0)))
    pad_idx = N + jnp.arange(EP - E, dtype=jnp.int32) % (NP - N)
    rows = jnp.concatenate([edge_index[0], pad_idx]).reshape(NW * NIROW, 128)
    cols1 = jnp.concatenate([edge_index[1], pad_idx])
    batchp = jnp.concatenate([batch, jnp.full((NP - N,), B, jnp.int32)])
    batchb = jnp.broadcast_to(batchp[:, None], (NP, B))

    embp = jnp.pad(emb, ((0, 0), (0, H - CED)))
    degp, cep = _sc_degree(cols1, embp, cell_lines)
    degt = degp.T

    hp1, dinvb = pl.pallas_call(
        _tc_scale_in,
        grid=(NTB,),
        in_specs=[_row_spec(), pl.BlockSpec((RB, NW), lambda i: (i, 0)),
                  _rep_spec((D, H))],
        out_specs=[_row_spec(), _row_spec()],
        out_shape=[jax.ShapeDtypeStruct((NP, H), f32)] * 2,
    )(xp, degt, gcn1_W)

    acc1 = _sc_edge_pass(hp1, rows, cols1)

    hp2 = pl.pallas_call(
        _tc_combine_mm,
        grid=(NTB,),
        in_specs=[pl.BlockSpec((2, RB, H), lambda i: (0, i, 0)),
                  _row_spec(), _row_spec(), _rep_spec((1, H)),
                  _rep_spec((H, H))],
        out_specs=_row_spec(),
        out_shape=jax.ShapeDtypeStruct((NP, H), f32),
    )(acc1, hp1, dinvb, gcn1_b[None, :], gcn2_W)

    acc2 = _sc_edge_pass(hp2, rows, cols1)

    combA = comb_W[:H]
    combB = jnp.pad(comb_W[H:], ((0, H - CED), (0, 0)))
    fc2_Wp = jnp.pad(fc2_W, ((0, 0), (0, LATP - LAT)))
    fc2_bp = jnp.pad(fc2_b, ((0, LATP - LAT),))

    out = pl.pallas_call(
        _tc_pool_head,
        grid=(NTB,),
        in_specs=[pl.BlockSpec((2, RB, H), lambda i: (0, i, 0)),
                  _row_spec(), _row_spec(), _rep_spec((1, H)),
                  pl.BlockSpec((RB, B), lambda i: (i, 0))] +
                 [_rep_spec(s) for s in
                  [(B, H), (H, H), (H, H), (1, H), (1, H), (1, H),
                   (H, H), (1, H), (1, H), (1, H), (H, LATP), (1, LATP)]],
        out_specs=pl.BlockSpec((B, LATP), lambda i: (0, 0)),
        out_shape=jax.ShapeDtypeStruct((B, LATP), f32),
        scratch_shapes=[pltpu.VMEM((B, H), f32)],
    )(acc2, hp2, dinvb, gcn2_b[None, :], batchb, cep, combA, combB,
      comb_b[None, :], lnc_w[None, :], lnc_b[None, :], fc1_W, fc1_b[None, :],
      ln1_w[None, :], ln1_b[None, :], fc2_Wp, fc2_bp[None, :])

    return out[:, :LAT]


# speedup vs baseline: 1.2163x; 1.2123x over previous
import functools

import jax
import jax.numpy as jnp
from jax import lax
from jax.experimental import pallas as pl
from jax.experimental.pallas import tpu as pltpu
from jax.experimental.pallas import tpu_sc as plsc

N = 10000
E = 320000
D = 128
H = 128
B = 64
NCL = 1000
CED = 64
LAT = 978

NW = 32
NP = 10240
EW = 10240
EP = NW * EW
CHUNK = 128
NSET = 2
NCHUNK = EW // CHUNK
NIROW = EW // 128
DCHUNK = 512
DNCHUNK = EW // DCHUNK
DNI = DNCHUNK // 2
TROWS = NP // 16
RB = 1024
NTB = NP // RB
LATP = 1024

_mesh = plsc.VectorSubcoreMesh(core_axis_name="c", subcore_axis_name="s")


@functools.partial(
    pl.kernel,
    out_type=[
        jax.ShapeDtypeStruct((NW, NP), jnp.float32),
        jax.ShapeDtypeStruct((B, H), jnp.float32),
    ],
    mesh=_mesh,
    scratch_types=[
        pltpu.VMEM((NP,), jnp.float32),
        pltpu.VMEM((DCHUNK,), jnp.int32),
        pltpu.VMEM((DCHUNK,), jnp.int32),
        pltpu.VMEM((B,), jnp.int32),
        pltpu.VMEM((B, H), jnp.float32),
        [pltpu.SemaphoreType.DMA for _ in range(3)],
    ],
    compiler_params=pltpu.CompilerParams(needs_layout_passes=False),
)
def _sc_degree(cols_hbm, emb_hbm, cl_hbm, deg_hbm, ce_hbm,
               histo, idxA, idxB, cl_v, ce_v, sems):
    c = lax.axis_index("c")
    s = lax.axis_index("s")
    w = c * 16 + s
    semA, semB, semE = sems

    def cstart(j, buf, sem):
        pltpu.async_copy(cols_hbm.at[pl.ds(w * EW + j * DCHUNK, DCHUNK)],
                         buf, sem)

    def cwait(buf, sem):
        pltpu.make_async_copy(cols_hbm.at[pl.ds(0, DCHUNK)], buf, sem).wait()

    cstart(0, idxA, semA)
    cstart(1, idxB, semB)

    def _zero(i, carry):
        histo[pl.ds(i * 16, 16)] = jnp.zeros((16,), jnp.float32)
        return carry
    lax.fori_loop(0, NP // 16, _zero, 0)

    ones16 = jnp.ones((16,), jnp.float32)

    def _step(i, carry):
        cwait(idxA, semA)
        for t in range(DCHUNK // 16):
            plsc.addupdate_scatter(histo, [idxA[pl.ds(t * 16, 16)]], ones16)

        @pl.when(i < DNI - 1)
        def _():
            cstart(2 * i + 2, idxA, semA)
        cwait(idxB, semB)
        for t in range(DCHUNK // 16):
            plsc.addupdate_scatter(histo, [idxB[pl.ds(t * 16, 16)]], ones16)

        @pl.when(i < DNI - 1)
        def _():
            cstart(2 * i + 3, idxB, semB)
        return carry
    lax.fori_loop(0, DNI, _step, 0)

    pltpu.sync_copy(histo, deg_hbm.at[w])

    @pl.when(w == 0)
    def _():
        pltpu.sync_copy(cl_hbm, cl_v)
        pltpu.async_copy(emb_hbm.at[cl_v], ce_v, semE).wait()
        pltpu.sync_copy(ce_v, ce_hbm)


NI = NCHUNK // NSET


@functools.partial(
    pl.kernel,
    out_type=jax.ShapeDtypeStruct((2, NP, H), jnp.float32),
    mesh=_mesh,
    scratch_types=[
        pltpu.VMEM_SHARED((NP, H), jnp.float32),
        pltpu.VMEM((NIROW, 128), jnp.int32),
        [pltpu.VMEM((CHUNK,), jnp.int32) for _ in range(NSET)],
        [pltpu.VMEM((CHUNK, H), jnp.float32) for _ in range(NSET)],
        [pltpu.SemaphoreType.DMA for _ in range(3 * NSET)],
    ],
)
def _sc_edge_pass(hp_hbm, rows_hbm, cols_hbm, acc_hbm,
                  acc_sp, idx_r, cbufs, gbufs, sems):
    c = lax.axis_index("c")
    s = lax.axis_index("s")
    w = c * 16 + s
    semg, sems_, semc = sems[:NSET], sems[NSET:2 * NSET], sems[2 * NSET:]

    pltpu.sync_copy(rows_hbm.at[pl.ds(w * NIROW, NIROW)], idx_r)

    z16 = jnp.zeros((16,), jnp.float32)

    def _fill(r, carry):
        for t in range(H // 16):
            gbufs[0][r, pl.ds(t * 16, 16)] = z16
        return carry
    lax.fori_loop(0, CHUNK, _fill, 0)

    def _zero(m, carry):
        pltpu.sync_copy(gbufs[0],
                        acc_sp.at[pl.ds(s * TROWS + m * CHUNK, CHUNK)])
        return carry
    lax.fori_loop(0, TROWS // CHUNK, _zero, 0)

    plsc.subcore_barrier()

    def gidx(i, b):
        row = (NSET * CHUNK // 128) * i + (b * CHUNK) // 128
        return idx_r.at[row, pl.ds((b * CHUNK) % 128, CHUNK)]

    def gstart(i, b):
        pltpu.async_copy(hp_hbm.at[gidx(i, b)], gbufs[b], semg[b])

    def gwait(b):
        pltpu.make_async_copy(hp_hbm.at[gidx(0, 0)], gbufs[b],
                              semg[b]).wait()

    def sstart(b):
        pltpu.async_copy(gbufs[b], acc_sp.at[cbufs[b]], sems_[b], add=True)

    def swait(b):
        pltpu.make_async_copy(gbufs[b], acc_sp.at[cbufs[b]], sems_[b]).wait()

    def cstart(i, b):
        k = NSET * i + b
        pltpu.async_copy(cols_hbm.at[pl.ds(w * EW + k * CHUNK, CHUNK)],
                         cbufs[b], semc[b])

    def cwait(b):
        pltpu.make_async_copy(cols_hbm.at[pl.ds(0, CHUNK)], cbufs[b],
                              semc[b]).wait()

    for b in range(NSET):
        cstart(0, b)
        gstart(0, b)

    def _body(i, carry):
        gwait(0)
        cwait(0)
        sstart(0)
        gwait(1)
        swait(0)

        @pl.when(i < NI - 1)
        def _():
            cstart(i + 1, 0)
            gstart(i + 1, 0)
        cwait(1)
        sstart(1)
        swait(1)

        @pl.when(i < NI - 1)
        def _():
            cstart(i + 1, 1)
            gstart(i + 1, 1)
        return carry
    lax.fori_loop(0, NI, _body, 0)

    plsc.subcore_barrier()
    pltpu.sync_copy(acc_sp.at[pl.ds(s * TROWS, TROWS)],
                    acc_hbm.at[c, pl.ds(s * TROWS, TROWS)])


def _tc_scale_in(x_ref, degt_ref, w1_ref, hp_ref, dinvb_ref):
    deg = jnp.sum(degt_ref[...], axis=1, keepdims=True) + 1.0
    dinvb = jnp.broadcast_to(lax.rsqrt(deg), (RB, H))
    z = jnp.dot(x_ref[...], w1_ref[...], preferred_element_type=jnp.float32)
    hp_ref[...] = dinvb * z
    dinvb_ref[...] = dinvb


def _tc_combine_mm(acc_ref, hp_ref, dinvb_ref, b_ref, w2_ref, hp2_ref):
    dinvb = dinvb_ref[...]
    u = jnp.maximum(
        dinvb * (acc_ref[0] + acc_ref[1] + hp_ref[...]) + b_ref[...], 0.0)
    hp2_ref[...] = dinvb * jnp.dot(u, w2_ref[...],
                                   preferred_element_type=jnp.float32)


def _ln(x, w, b, eps=1e-5):
    mu = jnp.mean(x, axis=-1, keepdims=True)
    var = jnp.mean((x - mu) ** 2, axis=-1, keepdims=True)
    return (x - mu) / jnp.sqrt(var + eps) * w + b


def _tc_pool_head(acc_ref, hp_ref, dinvb_ref, b_ref, batchb_ref, cep_ref,
                  combA_ref, combB_ref, comb_b_ref, lnc_w_ref, lnc_b_ref,
                  fc1_W_ref, fc1_b_ref, ln1_w_ref, ln1_b_ref, fc2_W_ref,
                  fc2_b_ref, out_ref, g_ref):
    h2 = jnp.maximum(
        dinvb_ref[...] * (acc_ref[0] + acc_ref[1] + hp_ref[...]) + b_ref[...],
        0.0)
    onehot = (batchb_ref[...] ==
              lax.broadcasted_iota(jnp.int32, (RB, B), 1)).astype(jnp.float32)
    part = lax.dot_general(onehot, h2, (((0,), (0,)), ((), ())),
                           preferred_element_type=jnp.float32)

    @pl.when(pl.program_id(0) == 0)
    def _():
        g_ref[...] = jnp.zeros_like(g_ref)
    g_ref[...] += part

    @pl.when(pl.program_id(0) == NTB - 1)
    def _():
        v = (jnp.dot(g_ref[...], combA_ref[...],
                     preferred_element_type=jnp.float32)
             + jnp.dot(cep_ref[...], combB_ref[...],
                       preferred_element_type=jnp.float32)
             + comb_b_ref[...])
        c1 = jnp.maximum(_ln(v, lnc_w_ref[...], lnc_b_ref[...]), 0.0)
        o = jnp.maximum(
            jnp.dot(c1, fc1_W_ref[...], preferred_element_type=jnp.float32)
            + fc1_b_ref[...], 0.0)
        o = _ln(o, ln1_w_ref[...], ln1_b_ref[...])
        out_ref[...] = (jnp.dot(o, fc2_W_ref[...],
                                preferred_element_type=jnp.float32)
                        + fc2_b_ref[...])


def _row_spec(nd=H):
    return pl.BlockSpec((RB, nd), lambda i: (i, 0))


def _rep_spec(shape):
    n = len(shape)
    return pl.BlockSpec(shape, lambda i, _n=n: (0,) * _n)


def kernel(x, edge_index, batch, cell_lines, gcn1_W, gcn1_b, gcn2_W, gcn2_b,
           emb, comb_W, comb_b, lnc_w, lnc_b, fc1_W, fc1_b, ln1_w, ln1_b,
           fc2_W, fc2_b):
    f32 = jnp.float32
    xp = jnp.pad(x, ((0, NP - N), (0, 0)))
    pad_idx = N + jnp.arange(EP - E, dtype=jnp.int32) % (NP - N)
    rows = jnp.concatenate([edge_index[0], pad_idx]).reshape(NW * NIROW, 128)
    cols1 = jnp.concatenate([edge_index[1], pad_idx])
    batchp = jnp.concatenate([batch, jnp.full((NP - N,), B, jnp.int32)])
    batchb = jnp.broadcast_to(batchp[:, None], (NP, B))

    embp = jnp.pad(emb, ((0, 0), (0, H - CED)))
    degp, cep = _sc_degree(cols1, embp, cell_lines)
    degt = degp.T

    hp1, dinvb = pl.pallas_call(
        _tc_scale_in,
        grid=(NTB,),
        in_specs=[_row_spec(), pl.BlockSpec((RB, NW), lambda i: (i, 0)),
                  _rep_spec((D, H))],
        out_specs=[_row_spec(), _row_spec()],
        out_shape=[jax.ShapeDtypeStruct((NP, H), f32)] * 2,
    )(xp, degt, gcn1_W)

    acc1 = _sc_edge_pass(hp1, rows, cols1)

    hp2 = pl.pallas_call(
        _tc_combine_mm,
        grid=(NTB,),
        in_specs=[pl.BlockSpec((2, RB, H), lambda i: (0, i, 0)),
                  _row_spec(), _row_spec(), _rep_spec((1, H)),
                  _rep_spec((H, H))],
        out_specs=_row_spec(),
        out_shape=jax.ShapeDtypeStruct((NP, H), f32),
    )(acc1, hp1, dinvb, gcn1_b[None, :], gcn2_W)

    acc2 = _sc_edge_pass(hp2, rows, cols1)

    combA = comb_W[:H]
    combB = jnp.pad(comb_W[H:], ((0, H - CED), (0, 0)))
    fc2_Wp = jnp.pad(fc2_W, ((0, 0), (0, LATP - LAT)))
    fc2_bp = jnp.pad(fc2_b, ((0, LATP - LAT),))

    out = pl.pallas_call(
        _tc_pool_head,
        grid=(NTB,),
        in_specs=[pl.BlockSpec((2, RB, H), lambda i: (0, i, 0)),
                  _row_spec(), _row_spec(), _rep_spec((1, H)),
                  pl.BlockSpec((RB, B), lambda i: (i, 0))] +
                 [_rep_spec(s) for s in
                  [(B, H), (H, H), (H, H), (1, H), (1, H), (1, H),
                   (H, H), (1, H), (1, H), (1, H), (H, LATP), (1, LATP)]],
        out_specs=pl.BlockSpec((B, LATP), lambda i: (0, 0)),
        out_shape=jax.ShapeDtypeStruct((B, LATP), f32),
        scratch_shapes=[pltpu.VMEM((B, H), f32)],
    )(acc2, hp2, dinvb, gcn2_b[None, :], batchb, cep, combA, combB,
      comb_b[None, :], lnc_w[None, :], lnc_b[None, :], fc1_W, fc1_b[None, :],
      ln1_w[None, :], ln1_b[None, :], fc2_Wp, fc2_bp[None, :])

    return out[:, :LAT]


# speedup vs baseline: 1.2672x; 1.0419x over previous
import functools

import jax
import jax.numpy as jnp
from jax import lax
from jax.experimental import pallas as pl
from jax.experimental.pallas import tpu as pltpu
from jax.experimental.pallas import tpu_sc as plsc

N = 10000
E = 320000
D = 128
H = 128
B = 64
NCL = 1000
CED = 64
LAT = 978

NW = 32
NP = 10240
EW = 10240
EP = NW * EW
CHUNK = 64
NSET = 4
NCHUNK = EW // CHUNK
NIROW = EW // 128
DCHUNK = 512
DNCHUNK = EW // DCHUNK
DNI = DNCHUNK // 2
TROWS = NP // 16
RB = 1024
NTB = NP // RB
LATP = 1024

_mesh = plsc.VectorSubcoreMesh(core_axis_name="c", subcore_axis_name="s")


@functools.partial(
    pl.kernel,
    out_type=[
        jax.ShapeDtypeStruct((NW, NP), jnp.float32),
        jax.ShapeDtypeStruct((B, H), jnp.float32),
    ],
    mesh=_mesh,
    scratch_types=[
        pltpu.VMEM((NP,), jnp.float32),
        pltpu.VMEM((DCHUNK,), jnp.int32),
        pltpu.VMEM((DCHUNK,), jnp.int32),
        pltpu.VMEM((B,), jnp.int32),
        pltpu.VMEM((B, H), jnp.float32),
        [pltpu.SemaphoreType.DMA for _ in range(3)],
    ],
    compiler_params=pltpu.CompilerParams(needs_layout_passes=False),
)
def _sc_degree(cols_hbm, emb_hbm, cl_hbm, deg_hbm, ce_hbm,
               histo, idxA, idxB, cl_v, ce_v, sems):
    c = lax.axis_index("c")
    s = lax.axis_index("s")
    w = c * 16 + s
    semA, semB, semE = sems

    def cstart(j, buf, sem):
        pltpu.async_copy(cols_hbm.at[pl.ds(w * EW + j * DCHUNK, DCHUNK)],
                         buf, sem)

    def cwait(buf, sem):
        pltpu.make_async_copy(cols_hbm.at[pl.ds(0, DCHUNK)], buf, sem).wait()

    cstart(0, idxA, semA)
    cstart(1, idxB, semB)

    def _zero(i, carry):
        histo[pl.ds(i * 16, 16)] = jnp.zeros((16,), jnp.float32)
        return carry
    lax.fori_loop(0, NP // 16, _zero, 0)

    ones16 = jnp.ones((16,), jnp.float32)

    def _step(i, carry):
        cwait(idxA, semA)
        for t in range(DCHUNK // 16):
            plsc.addupdate_scatter(histo, [idxA[pl.ds(t * 16, 16)]], ones16)

        @pl.when(i < DNI - 1)
        def _():
            cstart(2 * i + 2, idxA, semA)
        cwait(idxB, semB)
        for t in range(DCHUNK // 16):
            plsc.addupdate_scatter(histo, [idxB[pl.ds(t * 16, 16)]], ones16)

        @pl.when(i < DNI - 1)
        def _():
            cstart(2 * i + 3, idxB, semB)
        return carry
    lax.fori_loop(0, DNI, _step, 0)

    pltpu.sync_copy(histo, deg_hbm.at[w])

    @pl.when(w == 0)
    def _():
        pltpu.sync_copy(cl_hbm, cl_v)
        pltpu.async_copy(emb_hbm.at[cl_v], ce_v, semE).wait()
        pltpu.sync_copy(ce_v, ce_hbm)


NI = NCHUNK // NSET


@functools.partial(
    pl.kernel,
    out_type=jax.ShapeDtypeStruct((2, NP, H), jnp.float32),
    mesh=_mesh,
    scratch_types=[
        pltpu.VMEM_SHARED((NP, H), jnp.float32),
        pltpu.VMEM((NIROW, 128), jnp.int32),
        [pltpu.VMEM((CHUNK,), jnp.int32) for _ in range(NSET)],
        [pltpu.VMEM((CHUNK, H), jnp.float32) for _ in range(NSET)],
        [pltpu.SemaphoreType.DMA for _ in range(3 * NSET)],
    ],
)
def _sc_edge_pass(hp_hbm, rows_hbm, cols_hbm, acc_hbm,
                  acc_sp, idx_r, cbufs, gbufs, sems):
    c = lax.axis_index("c")
    s = lax.axis_index("s")
    w = c * 16 + s
    semg, sems_, semc = sems[:NSET], sems[NSET:2 * NSET], sems[2 * NSET:]

    pltpu.sync_copy(rows_hbm.at[pl.ds(w * NIROW, NIROW)], idx_r)

    z16 = jnp.zeros((16,), jnp.float32)

    def _fill(r, carry):
        for t in range(H // 16):
            gbufs[0][r, pl.ds(t * 16, 16)] = z16
        return carry
    lax.fori_loop(0, CHUNK, _fill, 0)

    def _zero(m, carry):
        pltpu.sync_copy(gbufs[0],
                        acc_sp.at[pl.ds(s * TROWS + m * CHUNK, CHUNK)])
        return carry
    lax.fori_loop(0, TROWS // CHUNK, _zero, 0)

    plsc.subcore_barrier()

    def gidx(i, b):
        row = (NSET * CHUNK // 128) * i + (b * CHUNK) // 128
        return idx_r.at[row, pl.ds((b * CHUNK) % 128, CHUNK)]

    def gstart(i, b):
        pltpu.async_copy(hp_hbm.at[gidx(i, b)], gbufs[b], semg[b])

    def gwait(b):
        pltpu.make_async_copy(hp_hbm.at[gidx(0, 0)], gbufs[b],
                              semg[b]).wait()

    def sstart(b):
        pltpu.async_copy(gbufs[b], acc_sp.at[cbufs[b]], sems_[b], add=True)

    def swait(b):
        pltpu.make_async_copy(gbufs[b], acc_sp.at[cbufs[b]], sems_[b]).wait()

    def cstart(i, b):
        k = NSET * i + b
        pltpu.async_copy(cols_hbm.at[pl.ds(w * EW + k * CHUNK, CHUNK)],
                         cbufs[b], semc[b])

    def cwait(b):
        pltpu.make_async_copy(cols_hbm.at[pl.ds(0, CHUNK)], cbufs[b],
                              semc[b]).wait()

    for b in range(NSET - 1):
        cstart(0, b)
        gstart(0, b)

    def _body(i, carry):
        for b in range(NSET):
            gwait(b)
            cwait(b)
            sstart(b)
            if b == 0:
                @pl.when(i > 0)
                def _():
                    swait(NSET - 1)
                cstart(i, NSET - 1)
                gstart(i, NSET - 1)
            else:
                swait(b - 1)

                @pl.when(i < NI - 1)
                def _(b=b):
                    cstart(i + 1, b - 1)
                    gstart(i + 1, b - 1)
        return carry
    lax.fori_loop(0, NI, _body, 0)
    swait(NSET - 1)

    plsc.subcore_barrier()
    pltpu.sync_copy(acc_sp.at[pl.ds(s * TROWS, TROWS)],
                    acc_hbm.at[c, pl.ds(s * TROWS, TROWS)])


def _tc_scale_in(x_ref, degt_ref, w1_ref, hp_ref, dinvb_ref):
    deg = jnp.sum(degt_ref[...], axis=1, keepdims=True) + 1.0
    dinvb = jnp.broadcast_to(lax.rsqrt(deg), (RB, H))
    z = jnp.dot(x_ref[...], w1_ref[...], preferred_element_type=jnp.float32)
    hp_ref[...] = dinvb * z
    dinvb_ref[...] = dinvb


def _tc_combine_mm(acc_ref, hp_ref, dinvb_ref, b_ref, w2_ref, hp2_ref):
    dinvb = dinvb_ref[...]
    u = jnp.maximum(
        dinvb * (acc_ref[0] + acc_ref[1] + hp_ref[...]) + b_ref[...], 0.0)
    hp2_ref[...] = dinvb * jnp.dot(u, w2_ref[...],
                                   preferred_element_type=jnp.float32)


def _ln(x, w, b, eps=1e-5):
    mu = jnp.mean(x, axis=-1, keepdims=True)
    var = jnp.mean((x - mu) ** 2, axis=-1, keepdims=True)
    return (x - mu) / jnp.sqrt(var + eps) * w + b


def _tc_pool_head(acc_ref, hp_ref, dinvb_ref, b_ref, batchb_ref, cep_ref,
                  combA_ref, combB_ref, comb_b_ref, lnc_w_ref, lnc_b_ref,
                  fc1_W_ref, fc1_b_ref, ln1_w_ref, ln1_b_ref, fc2_W_ref,
                  fc2_b_ref, out_ref, g_ref):
    h2 = jnp.maximum(
        dinvb_ref[...] * (acc_ref[0] + acc_ref[1] + hp_ref[...]) + b_ref[...],
        0.0)
    onehot = (batchb_ref[...] ==
              lax.broadcasted_iota(jnp.int32, (RB, B), 1)).astype(jnp.float32)
    part = lax.dot_general(onehot, h2, (((0,), (0,)), ((), ())),
                           preferred_element_type=jnp.float32)

    @pl.when(pl.program_id(0) == 0)
    def _():
        g_ref[...] = jnp.zeros_like(g_ref)
    g_ref[...] += part

    @pl.when(pl.program_id(0) == NTB - 1)
    def _():
        v = (jnp.dot(g_ref[...], combA_ref[...],
                     preferred_element_type=jnp.float32)
             + jnp.dot(cep_ref[...], combB_ref[...],
                       preferred_element_type=jnp.float32)
             + comb_b_ref[...])
        c1 = jnp.maximum(_ln(v, lnc_w_ref[...], lnc_b_ref[...]), 0.0)
        o = jnp.maximum(
            jnp.dot(c1, fc1_W_ref[...], preferred_element_type=jnp.float32)
            + fc1_b_ref[...], 0.0)
        o = _ln(o, ln1_w_ref[...], ln1_b_ref[...])
        out_ref[...] = (jnp.dot(o, fc2_W_ref[...],
                                preferred_element_type=jnp.float32)
                        + fc2_b_ref[...])


def _row_spec(nd=H):
    return pl.BlockSpec((RB, nd), lambda i: (i, 0))


def _rep_spec(shape):
    n = len(shape)
    return pl.BlockSpec(shape, lambda i, _n=n: (0,) * _n)


def kernel(x, edge_index, batch, cell_lines, gcn1_W, gcn1_b, gcn2_W, gcn2_b,
           emb, comb_W, comb_b, lnc_w, lnc_b, fc1_W, fc1_b, ln1_w, ln1_b,
           fc2_W, fc2_b):
    f32 = jnp.float32
    xp = jnp.pad(x, ((0, NP - N), (0, 0)))
    pad_idx = N + jnp.arange(EP - E, dtype=jnp.int32) % (NP - N)
    rows = jnp.concatenate([edge_index[0], pad_idx]).reshape(NW * NIROW, 128)
    cols1 = jnp.concatenate([edge_index[1], pad_idx])
    batchp = jnp.concatenate([batch, jnp.full((NP - N,), B, jnp.int32)])
    batchb = jnp.broadcast_to(batchp[:, None], (NP, B))

    embp = jnp.pad(emb, ((0, 0), (0, H - CED)))
    degp, cep = _sc_degree(cols1, embp, cell_lines)
    degt = degp.T

    hp1, dinvb = pl.pallas_call(
        _tc_scale_in,
        grid=(NTB,),
        in_specs=[_row_spec(), pl.BlockSpec((RB, NW), lambda i: (i, 0)),
                  _rep_spec((D, H))],
        out_specs=[_row_spec(), _row_spec()],
        out_shape=[jax.ShapeDtypeStruct((NP, H), f32)] * 2,
    )(xp, degt, gcn1_W)

    acc1 = _sc_edge_pass(hp1, rows, cols1)

    hp2 = pl.pallas_call(
        _tc_combine_mm,
        grid=(NTB,),
        in_specs=[pl.BlockSpec((2, RB, H), lambda i: (0, i, 0)),
                  _row_spec(), _row_spec(), _rep_spec((1, H)),
                  _rep_spec((H, H))],
        out_specs=_row_spec(),
        out_shape=jax.ShapeDtypeStruct((NP, H), f32),
    )(acc1, hp1, dinvb, gcn1_b[None, :], gcn2_W)

    acc2 = _sc_edge_pass(hp2, rows, cols1)

    combA = comb_W[:H]
    combB = jnp.pad(comb_W[H:], ((0, H - CED), (0, 0)))
    fc2_Wp = jnp.pad(fc2_W, ((0, 0), (0, LATP - LAT)))
    fc2_bp = jnp.pad(fc2_b, ((0, LATP - LAT),))

    out = pl.pallas_call(
        _tc_pool_head,
        grid=(NTB,),
        in_specs=[pl.BlockSpec((2, RB, H), lambda i: (0, i, 0)),
                  _row_spec(), _row_spec(), _rep_spec((1, H)),
                  pl.BlockSpec((RB, B), lambda i: (i, 0))] +
                 [_rep_spec(s) for s in
                  [(B, H), (H, H), (H, H), (1, H), (1, H), (1, H),
                   (H, H), (1, H), (1, H), (1, H), (H, LATP), (1, LATP)]],
        out_specs=pl.BlockSpec((B, LATP), lambda i: (0, 0)),
        out_shape=jax.ShapeDtypeStruct((B, LATP), f32),
        scratch_shapes=[pltpu.VMEM((B, H), f32)],
    )(acc2, hp2, dinvb, gcn2_b[None, :], batchb, cep, combA, combB,
      comb_b[None, :], lnc_w[None, :], lnc_b[None, :], fc1_W, fc1_b[None, :],
      ln1_w[None, :], ln1_b[None, :], fc2_Wp, fc2_bp[None, :])

    return out[:, :LAT]


# speedup vs baseline: 1.2729x; 1.0045x over previous
import functools

import jax
import jax.numpy as jnp
from jax import lax
from jax.experimental import pallas as pl
from jax.experimental.pallas import tpu as pltpu
from jax.experimental.pallas import tpu_sc as plsc

N = 10000
E = 320000
D = 128
H = 128
B = 64
NCL = 1000
CED = 64
LAT = 978

NW = 32
NP = 10240
EW = 10240
EP = NW * EW
CHUNK = 64
NSET = 4
NCHUNK = EW // CHUNK
NIROW = EW // 128
DCHUNK = 512
DNCHUNK = EW // DCHUNK
DNI = DNCHUNK // 2
TROWS = NP // 16
RB = 1024
NTB = NP // RB
LATP = 1024

_mesh = plsc.VectorSubcoreMesh(core_axis_name="c", subcore_axis_name="s")


@functools.partial(
    pl.kernel,
    out_type=[
        jax.ShapeDtypeStruct((NW, NP), jnp.float32),
        jax.ShapeDtypeStruct((B, H), jnp.float32),
    ],
    mesh=_mesh,
    scratch_types=[
        pltpu.VMEM((NP,), jnp.float32),
        pltpu.VMEM((DCHUNK,), jnp.int32),
        pltpu.VMEM((DCHUNK,), jnp.int32),
        pltpu.VMEM((B,), jnp.int32),
        pltpu.VMEM((B, H), jnp.float32),
        [pltpu.SemaphoreType.DMA for _ in range(3)],
    ],
    compiler_params=pltpu.CompilerParams(needs_layout_passes=False),
)
def _sc_degree(cols_hbm, emb_hbm, cl_hbm, deg_hbm, ce_hbm,
               histo, idxA, idxB, cl_v, ce_v, sems):
    c = lax.axis_index("c")
    s = lax.axis_index("s")
    w = c * 16 + s
    semA, semB, semE = sems

    def cstart(j, buf, sem):
        pltpu.async_copy(cols_hbm.at[pl.ds(w * EW + j * DCHUNK, DCHUNK)],
                         buf, sem)

    def cwait(buf, sem):
        pltpu.make_async_copy(cols_hbm.at[pl.ds(0, DCHUNK)], buf, sem).wait()

    cstart(0, idxA, semA)
    cstart(1, idxB, semB)

    def _zero(i, carry):
        histo[pl.ds(i * 16, 16)] = jnp.zeros((16,), jnp.float32)
        return carry
    lax.fori_loop(0, NP // 16, _zero, 0)

    ones16 = jnp.ones((16,), jnp.float32)

    def _step(i, carry):
        cwait(idxA, semA)
        for t in range(DCHUNK // 16):
            plsc.addupdate_scatter(histo, [idxA[pl.ds(t * 16, 16)]], ones16)

        @pl.when(i < DNI - 1)
        def _():
            cstart(2 * i + 2, idxA, semA)
        cwait(idxB, semB)
        for t in range(DCHUNK // 16):
            plsc.addupdate_scatter(histo, [idxB[pl.ds(t * 16, 16)]], ones16)

        @pl.when(i < DNI - 1)
        def _():
            cstart(2 * i + 3, idxB, semB)
        return carry
    lax.fori_loop(0, DNI, _step, 0)

    pltpu.sync_copy(histo, deg_hbm.at[w])

    @pl.when(w == 0)
    def _():
        pltpu.sync_copy(cl_hbm, cl_v)
        pltpu.async_copy(emb_hbm.at[cl_v], ce_v, semE).wait()
        pltpu.sync_copy(ce_v, ce_hbm)


NI = NCHUNK // NSET


@functools.partial(
    pl.kernel,
    out_type=jax.ShapeDtypeStruct((2, NP, H), jnp.float32),
    mesh=_mesh,
    scratch_types=[
        pltpu.VMEM_SHARED((NP, H), jnp.float32),
        pltpu.VMEM((NIROW, 128), jnp.int32),
        [pltpu.VMEM((CHUNK,), jnp.int32) for _ in range(NSET)],
        [pltpu.VMEM((CHUNK, H), jnp.float32) for _ in range(NSET)],
        [pltpu.SemaphoreType.DMA for _ in range(3 * NSET)],
    ],
)
def _sc_edge_pass(hp_hbm, rows_hbm, cols_hbm, acc_hbm,
                  acc_sp, idx_r, cbufs, gbufs, sems):
    c = lax.axis_index("c")
    s = lax.axis_index("s")
    w = c * 16 + s
    semg, sems_, semc = sems[:NSET], sems[NSET:2 * NSET], sems[2 * NSET:]

    pltpu.sync_copy(rows_hbm.at[pl.ds(w * NIROW, NIROW)], idx_r)

    z16 = jnp.zeros((16,), jnp.float32)

    def _fill(r, carry):
        for t in range(H // 16):
            gbufs[0][r, pl.ds(t * 16, 16)] = z16
        return carry
    lax.fori_loop(0, CHUNK, _fill, 0)

    def _zero(m, carry):
        pltpu.sync_copy(gbufs[0],
                        acc_sp.at[pl.ds(s * TROWS + m * CHUNK, CHUNK)])
        return carry
    lax.fori_loop(0, TROWS // CHUNK, _zero, 0)

    plsc.subcore_barrier()

    def gidx(i, b):
        row = (NSET * CHUNK // 128) * i + (b * CHUNK) // 128
        return idx_r.at[row, pl.ds((b * CHUNK) % 128, CHUNK)]

    def gstart(i, b):
        pltpu.async_copy(hp_hbm.at[gidx(i, b)], gbufs[b], semg[b])

    def gwait(b):
        pltpu.make_async_copy(hp_hbm.at[gidx(0, 0)], gbufs[b],
                              semg[b]).wait()

    def sstart(b):
        pltpu.async_copy(gbufs[b], acc_sp.at[cbufs[b]], sems_[b], add=True)

    def swait(b):
        pltpu.make_async_copy(gbufs[b], acc_sp.at[cbufs[b]], sems_[b]).wait()

    def cstart(i, b):
        k = NSET * i + b
        pltpu.async_copy(cols_hbm.at[pl.ds(w * EW + k * CHUNK, CHUNK)],
                         cbufs[b], semc[b])

    def cwait(b):
        pltpu.make_async_copy(cols_hbm.at[pl.ds(0, CHUNK)], cbufs[b],
                              semc[b]).wait()

    for b in range(NSET - 1):
        cstart(0, b)
        gstart(0, b)

    def _body(i, carry):
        for b in range(NSET):
            gwait(b)
            cwait(b)
            sstart(b)
            if b == 0:
                @pl.when(i > 0)
                def _():
                    swait(NSET - 1)
                cstart(i, NSET - 1)
                gstart(i, NSET - 1)
            else:
                swait(b - 1)

                @pl.when(i < NI - 1)
                def _(b=b):
                    cstart(i + 1, b - 1)
                    gstart(i + 1, b - 1)
        return carry
    lax.fori_loop(0, NI, _body, 0)
    swait(NSET - 1)

    plsc.subcore_barrier()
    pltpu.sync_copy(acc_sp.at[pl.ds(s * TROWS, TROWS)],
                    acc_hbm.at[c, pl.ds(s * TROWS, TROWS)])


def _dinvb(degt_ref):
    deg = jnp.sum(degt_ref[...], axis=1, keepdims=True) + 1.0
    return jnp.broadcast_to(lax.rsqrt(deg), (RB, H))


def _tc_scale_in(x_ref, degt_ref, w1_ref, hp_ref):
    z = jnp.dot(x_ref[...], w1_ref[...], preferred_element_type=jnp.float32)
    hp_ref[...] = _dinvb(degt_ref) * z


def _tc_combine_mm(acc_ref, hp_ref, degt_ref, b_ref, w2_ref, hp2_ref):
    dinvb = _dinvb(degt_ref)
    u = jnp.maximum(
        dinvb * (acc_ref[0] + acc_ref[1] + hp_ref[...]) + b_ref[...], 0.0)
    hp2_ref[...] = dinvb * jnp.dot(u, w2_ref[...],
                                   preferred_element_type=jnp.float32)


def _ln(x, w, b, eps=1e-5):
    mu = jnp.mean(x, axis=-1, keepdims=True)
    var = jnp.mean((x - mu) ** 2, axis=-1, keepdims=True)
    return (x - mu) / jnp.sqrt(var + eps) * w + b


def _tc_pool_head(acc_ref, hp_ref, degt_ref, b_ref, batchb_ref, cep_ref,
                  combA_ref, combB_ref, comb_b_ref, lnc_w_ref, lnc_b_ref,
                  fc1_W_ref, fc1_b_ref, ln1_w_ref, ln1_b_ref, fc2_W_ref,
                  fc2_b_ref, out_ref, g_ref):
    h2 = jnp.maximum(
        _dinvb(degt_ref) * (acc_ref[0] + acc_ref[1] + hp_ref[...])
        + b_ref[...], 0.0)
    onehot = (batchb_ref[...] ==
              lax.broadcasted_iota(jnp.int32, (RB, B), 1)).astype(jnp.float32)
    part = lax.dot_general(onehot, h2, (((0,), (0,)), ((), ())),
                           preferred_element_type=jnp.float32)

    @pl.when(pl.program_id(0) == 0)
    def _():
        g_ref[...] = jnp.zeros_like(g_ref)
    g_ref[...] += part

    @pl.when(pl.program_id(0) == NTB - 1)
    def _():
        v = (jnp.dot(g_ref[...], combA_ref[...],
                     preferred_element_type=jnp.float32)
             + jnp.dot(cep_ref[...], combB_ref[...],
                       preferred_element_type=jnp.float32)
             + comb_b_ref[...])
        c1 = jnp.maximum(_ln(v, lnc_w_ref[...], lnc_b_ref[...]), 0.0)
        o = jnp.maximum(
            jnp.dot(c1, fc1_W_ref[...], preferred_element_type=jnp.float32)
            + fc1_b_ref[...], 0.0)
        o = _ln(o, ln1_w_ref[...], ln1_b_ref[...])
        out_ref[...] = (jnp.dot(o, fc2_W_ref[...],
                                preferred_element_type=jnp.float32)
                        + fc2_b_ref[...])


def _row_spec(nd=H):
    return pl.BlockSpec((RB, nd), lambda i: (i, 0))


def _rep_spec(shape):
    n = len(shape)
    return pl.BlockSpec(shape, lambda i, _n=n: (0,) * _n)


def kernel(x, edge_index, batch, cell_lines, gcn1_W, gcn1_b, gcn2_W, gcn2_b,
           emb, comb_W, comb_b, lnc_w, lnc_b, fc1_W, fc1_b, ln1_w, ln1_b,
           fc2_W, fc2_b):
    f32 = jnp.float32
    xp = jnp.pad(x, ((0, NP - N), (0, 0)))
    pad_idx = N + jnp.arange(EP - E, dtype=jnp.int32) % (NP - N)
    rows = jnp.concatenate([edge_index[0], pad_idx]).reshape(NW * NIROW, 128)
    cols1 = jnp.concatenate([edge_index[1], pad_idx])
    batchp = jnp.concatenate([batch, jnp.full((NP - N,), B, jnp.int32)])
    batchb = jnp.broadcast_to(batchp[:, None], (NP, B))

    embp = jnp.pad(emb, ((0, 0), (0, H - CED)))
    degp, cep = _sc_degree(cols1, embp, cell_lines)
    degt = degp.T

    degt_spec = pl.BlockSpec((RB, NW), lambda i: (i, 0))
    hp1 = pl.pallas_call(
        _tc_scale_in,
        grid=(NTB,),
        in_specs=[_row_spec(), degt_spec, _rep_spec((D, H))],
        out_specs=_row_spec(),
        out_shape=jax.ShapeDtypeStruct((NP, H), f32),
    )(xp, degt, gcn1_W)

    acc1 = _sc_edge_pass(hp1, rows, cols1)

    hp2 = pl.pallas_call(
        _tc_combine_mm,
        grid=(NTB,),
        in_specs=[pl.BlockSpec((2, RB, H), lambda i: (0, i, 0)),
                  _row_spec(), degt_spec, _rep_spec((1, H)),
                  _rep_spec((H, H))],
        out_specs=_row_spec(),
        out_shape=jax.ShapeDtypeStruct((NP, H), f32),
    )(acc1, hp1, degt, gcn1_b[None, :], gcn2_W)

    acc2 = _sc_edge_pass(hp2, rows, cols1)

    combA = comb_W[:H]
    combB = jnp.pad(comb_W[H:], ((0, H - CED), (0, 0)))
    fc2_Wp = jnp.pad(fc2_W, ((0, 0), (0, LATP - LAT)))
    fc2_bp = jnp.pad(fc2_b, ((0, LATP - LAT),))

    out = pl.pallas_call(
        _tc_pool_head,
        grid=(NTB,),
        in_specs=[pl.BlockSpec((2, RB, H), lambda i: (0, i, 0)),
                  _row_spec(), degt_spec, _rep_spec((1, H)),
                  pl.BlockSpec((RB, B), lambda i: (i, 0))] +
                 [_rep_spec(s) for s in
                  [(B, H), (H, H), (H, H), (1, H), (1, H), (1, H),
                   (H, H), (1, H), (1, H), (1, H), (H, LATP), (1, LATP)]],
        out_specs=pl.BlockSpec((B, LATP), lambda i: (0, 0)),
        out_shape=jax.ShapeDtypeStruct((B, LATP), f32),
        scratch_shapes=[pltpu.VMEM((B, H), f32)],
    )(acc2, hp2, degt, gcn2_b[None, :], batchb, cep, combA, combB,
      comb_b[None, :], lnc_w[None, :], lnc_b[None, :], fc1_W, fc1_b[None, :],
      ln1_w[None, :], ln1_b[None, :], fc2_Wp, fc2_bp[None, :])

    return out[:, :LAT]


# speedup vs baseline: 1.3078x; 1.0274x over previous
import functools

import jax
import jax.numpy as jnp
from jax import lax
from jax.experimental import pallas as pl
from jax.experimental.pallas import tpu as pltpu
from jax.experimental.pallas import tpu_sc as plsc

N = 10000
E = 320000
D = 128
H = 128
B = 64
NCL = 1000
CED = 64
LAT = 978

NW = 32
NP = 10240
EW = 10240
EP = NW * EW
CHUNK = 64
NSET = 4
NCHUNK = EW // CHUNK
NIROW = EW // 128
DCHUNK = 512
DNCHUNK = EW // DCHUNK
DNI = DNCHUNK // 2
TROWS = NP // 16
RB = 2048
NTB = NP // RB
LATP = 1024

_mesh = plsc.VectorSubcoreMesh(core_axis_name="c", subcore_axis_name="s")


@functools.partial(
    pl.kernel,
    out_type=[
        jax.ShapeDtypeStruct((NW, NP), jnp.float32),
        jax.ShapeDtypeStruct((B, H), jnp.float32),
    ],
    mesh=_mesh,
    scratch_types=[
        pltpu.VMEM((NP,), jnp.float32),
        pltpu.VMEM((DCHUNK,), jnp.int32),
        pltpu.VMEM((DCHUNK,), jnp.int32),
        pltpu.VMEM((B,), jnp.int32),
        pltpu.VMEM((B, H), jnp.float32),
        [pltpu.SemaphoreType.DMA for _ in range(3)],
    ],
    compiler_params=pltpu.CompilerParams(needs_layout_passes=False),
)
def _sc_degree(cols_hbm, emb_hbm, cl_hbm, deg_hbm, ce_hbm,
               histo, idxA, idxB, cl_v, ce_v, sems):
    c = lax.axis_index("c")
    s = lax.axis_index("s")
    w = c * 16 + s
    semA, semB, semE = sems

    def cstart(j, buf, sem):
        pltpu.async_copy(cols_hbm.at[pl.ds(w * EW + j * DCHUNK, DCHUNK)],
                         buf, sem)

    def cwait(buf, sem):
        pltpu.make_async_copy(cols_hbm.at[pl.ds(0, DCHUNK)], buf, sem).wait()

    cstart(0, idxA, semA)
    cstart(1, idxB, semB)

    def _zero(i, carry):
        histo[pl.ds(i * 16, 16)] = jnp.zeros((16,), jnp.float32)
        return carry
    lax.fori_loop(0, NP // 16, _zero, 0)

    ones16 = jnp.ones((16,), jnp.float32)

    def _step(i, carry):
        cwait(idxA, semA)
        for t in range(DCHUNK // 16):
            plsc.addupdate_scatter(histo, [idxA[pl.ds(t * 16, 16)]], ones16)

        @pl.when(i < DNI - 1)
        def _():
            cstart(2 * i + 2, idxA, semA)
        cwait(idxB, semB)
        for t in range(DCHUNK // 16):
            plsc.addupdate_scatter(histo, [idxB[pl.ds(t * 16, 16)]], ones16)

        @pl.when(i < DNI - 1)
        def _():
            cstart(2 * i + 3, idxB, semB)
        return carry
    lax.fori_loop(0, DNI, _step, 0)

    pltpu.sync_copy(histo, deg_hbm.at[w])

    @pl.when(w == 0)
    def _():
        pltpu.sync_copy(cl_hbm, cl_v)
        pltpu.async_copy(emb_hbm.at[cl_v], ce_v, semE).wait()
        pltpu.sync_copy(ce_v, ce_hbm)


NI = NCHUNK // NSET


@functools.partial(
    pl.kernel,
    out_type=jax.ShapeDtypeStruct((2, NP, H), jnp.float32),
    mesh=_mesh,
    scratch_types=[
        pltpu.VMEM_SHARED((NP, H), jnp.float32),
        pltpu.VMEM((NIROW, 128), jnp.int32),
        [pltpu.VMEM((CHUNK,), jnp.int32) for _ in range(NSET)],
        [pltpu.VMEM((CHUNK, H), jnp.float32) for _ in range(NSET)],
        [pltpu.SemaphoreType.DMA for _ in range(3 * NSET)],
    ],
)
def _sc_edge_pass(hp_hbm, rows_hbm, cols_hbm, acc_hbm,
                  acc_sp, idx_r, cbufs, gbufs, sems):
    c = lax.axis_index("c")
    s = lax.axis_index("s")
    w = c * 16 + s
    semg, sems_, semc = sems[:NSET], sems[NSET:2 * NSET], sems[2 * NSET:]

    pltpu.sync_copy(rows_hbm.at[pl.ds(w * NIROW, NIROW)], idx_r)

    z16 = jnp.zeros((16,), jnp.float32)

    def _fill(r, carry):
        for t in range(H // 16):
            gbufs[0][r, pl.ds(t * 16, 16)] = z16
        return carry
    lax.fori_loop(0, CHUNK, _fill, 0)

    def _zero(m, carry):
        pltpu.sync_copy(gbufs[0],
                        acc_sp.at[pl.ds(s * TROWS + m * CHUNK, CHUNK)])
        return carry
    lax.fori_loop(0, TROWS // CHUNK, _zero, 0)

    plsc.subcore_barrier()

    def gidx(i, b):
        row = (NSET * CHUNK // 128) * i + (b * CHUNK) // 128
        return idx_r.at[row, pl.ds((b * CHUNK) % 128, CHUNK)]

    def gstart(i, b):
        pltpu.async_copy(hp_hbm.at[gidx(i, b)], gbufs[b], semg[b])

    def gwait(b):
        pltpu.make_async_copy(hp_hbm.at[gidx(0, 0)], gbufs[b],
                              semg[b]).wait()

    def sstart(b):
        pltpu.async_copy(gbufs[b], acc_sp.at[cbufs[b]], sems_[b], add=True)

    def swait(b):
        pltpu.make_async_copy(gbufs[b], acc_sp.at[cbufs[b]], sems_[b]).wait()

    def cstart(i, b):
        k = NSET * i + b
        pltpu.async_copy(cols_hbm.at[pl.ds(w * EW + k * CHUNK, CHUNK)],
                         cbufs[b], semc[b])

    def cwait(b):
        pltpu.make_async_copy(cols_hbm.at[pl.ds(0, CHUNK)], cbufs[b],
                              semc[b]).wait()

    for b in range(NSET - 1):
        cstart(0, b)
        gstart(0, b)

    def _body(i, carry):
        for b in range(NSET):
            gwait(b)
            cwait(b)
            sstart(b)
            if b == 0:
                @pl.when(i > 0)
                def _():
                    swait(NSET - 1)
                cstart(i, NSET - 1)
                gstart(i, NSET - 1)
            else:
                swait(b - 1)

                @pl.when(i < NI - 1)
                def _(b=b):
                    cstart(i + 1, b - 1)
                    gstart(i + 1, b - 1)
        return carry
    lax.fori_loop(0, NI, _body, 0)
    swait(NSET - 1)

    plsc.subcore_barrier()
    pltpu.sync_copy(acc_sp.at[pl.ds(s * TROWS, TROWS)],
                    acc_hbm.at[c, pl.ds(s * TROWS, TROWS)])


def _dinvb(degt_ref):
    deg = jnp.sum(degt_ref[...], axis=1, keepdims=True) + 1.0
    return jnp.broadcast_to(lax.rsqrt(deg), (RB, H))


def _tc_scale_in(x_ref, degt_ref, w1_ref, hp_ref):
    z = jnp.dot(x_ref[...], w1_ref[...], preferred_element_type=jnp.float32)
    hp_ref[...] = _dinvb(degt_ref) * z


def _tc_combine_mm(acc_ref, hp_ref, degt_ref, b_ref, w2_ref, hp2_ref):
    dinvb = _dinvb(degt_ref)
    u = jnp.maximum(
        dinvb * (acc_ref[0] + acc_ref[1] + hp_ref[...]) + b_ref[...], 0.0)
    hp2_ref[...] = dinvb * jnp.dot(u, w2_ref[...],
                                   preferred_element_type=jnp.float32)


def _ln(x, w, b, eps=1e-5):
    mu = jnp.mean(x, axis=-1, keepdims=True)
    var = jnp.mean((x - mu) ** 2, axis=-1, keepdims=True)
    return (x - mu) / jnp.sqrt(var + eps) * w + b


def _tc_pool_head(acc_ref, hp_ref, degt_ref, b_ref, batchb_ref, cep_ref,
                  combA_ref, combB_ref, comb_b_ref, lnc_w_ref, lnc_b_ref,
                  fc1_W_ref, fc1_b_ref, ln1_w_ref, ln1_b_ref, fc2_W_ref,
                  fc2_b_ref, out_ref, g_ref):
    h2 = jnp.maximum(
        _dinvb(degt_ref) * (acc_ref[0] + acc_ref[1] + hp_ref[...])
        + b_ref[...], 0.0)
    onehot = (batchb_ref[...] ==
              lax.broadcasted_iota(jnp.int32, (RB, B), 1)).astype(jnp.float32)
    part = lax.dot_general(onehot, h2, (((0,), (0,)), ((), ())),
                           preferred_element_type=jnp.float32)

    @pl.when(pl.program_id(0) == 0)
    def _():
        g_ref[...] = jnp.zeros_like(g_ref)
    g_ref[...] += part

    @pl.when(pl.program_id(0) == NTB - 1)
    def _():
        v = (jnp.dot(g_ref[...], combA_ref[...],
                     preferred_element_type=jnp.float32)
             + jnp.dot(cep_ref[...], combB_ref[...],
                       preferred_element_type=jnp.float32)
             + comb_b_ref[...])
        c1 = jnp.maximum(_ln(v, lnc_w_ref[...], lnc_b_ref[...]), 0.0)
        o = jnp.maximum(
            jnp.dot(c1, fc1_W_ref[...], preferred_element_type=jnp.float32)
            + fc1_b_ref[...], 0.0)
        o = _ln(o, ln1_w_ref[...], ln1_b_ref[...])
        out_ref[...] = (jnp.dot(o, fc2_W_ref[...],
                                preferred_element_type=jnp.float32)
                        + fc2_b_ref[...])


def _row_spec(nd=H):
    return pl.BlockSpec((RB, nd), lambda i: (i, 0))


def _rep_spec(shape):
    n = len(shape)
    return pl.BlockSpec(shape, lambda i, _n=n: (0,) * _n)


def kernel(x, edge_index, batch, cell_lines, gcn1_W, gcn1_b, gcn2_W, gcn2_b,
           emb, comb_W, comb_b, lnc_w, lnc_b, fc1_W, fc1_b, ln1_w, ln1_b,
           fc2_W, fc2_b):
    f32 = jnp.float32
    xp = jnp.pad(x, ((0, NP - N), (0, 0)))
    pad_idx = N + jnp.arange(EP - E, dtype=jnp.int32) % (NP - N)
    rows = jnp.concatenate([edge_index[0], pad_idx]).reshape(NW * NIROW, 128)
    cols1 = jnp.concatenate([edge_index[1], pad_idx])
    batchp = jnp.concatenate([batch, jnp.full((NP - N,), B, jnp.int32)])
    batchb = jnp.broadcast_to(batchp[:, None], (NP, B))

    embp = jnp.pad(emb, ((0, 0), (0, H - CED)))
    degp, cep = _sc_degree(cols1, embp, cell_lines)
    degt = degp.T

    degt_spec = pl.BlockSpec((RB, NW), lambda i: (i, 0))
    hp1 = pl.pallas_call(
        _tc_scale_in,
        grid=(NTB,),
        in_specs=[_row_spec(), degt_spec, _rep_spec((D, H))],
        out_specs=_row_spec(),
        out_shape=jax.ShapeDtypeStruct((NP, H), f32),
    )(xp, degt, gcn1_W)

    acc1 = _sc_edge_pass(hp1, rows, cols1)

    hp2 = pl.pallas_call(
        _tc_combine_mm,
        grid=(NTB,),
        in_specs=[pl.BlockSpec((2, RB, H), lambda i: (0, i, 0)),
                  _row_spec(), degt_spec, _rep_spec((1, H)),
                  _rep_spec((H, H))],
        out_specs=_row_spec(),
        out_shape=jax.ShapeDtypeStruct((NP, H), f32),
    )(acc1, hp1, degt, gcn1_b[None, :], gcn2_W)

    acc2 = _sc_edge_pass(hp2, rows, cols1)

    combA = comb_W[:H]
    combB = jnp.pad(comb_W[H:], ((0, H - CED), (0, 0)))
    fc2_Wp = jnp.pad(fc2_W, ((0, 0), (0, LATP - LAT)))
    fc2_bp = jnp.pad(fc2_b, ((0, LATP - LAT),))

    out = pl.pallas_call(
        _tc_pool_head,
        grid=(NTB,),
        in_specs=[pl.BlockSpec((2, RB, H), lambda i: (0, i, 0)),
                  _row_spec(), degt_spec, _rep_spec((1, H)),
                  pl.BlockSpec((RB, B), lambda i: (i, 0))] +
                 [_rep_spec(s) for s in
                  [(B, H), (H, H), (H, H), (1, H), (1, H), (1, H),
                   (H, H), (1, H), (1, H), (1, H), (H, LATP), (1, LATP)]],
        out_specs=pl.BlockSpec((B, LATP), lambda i: (0, 0)),
        out_shape=jax.ShapeDtypeStruct((B, LATP), f32),
        scratch_shapes=[pltpu.VMEM((B, H), f32)],
    )(acc2, hp2, degt, gcn2_b[None, :], batchb, cep, combA, combB,
      comb_b[None, :], lnc_w[None, :], lnc_b[None, :], fc1_W, fc1_b[None, :],
      ln1_w[None, :], ln1_b[None, :], fc2_Wp, fc2_bp[None, :])

    return out[:, :LAT]
